# baseline (speedup 1.0000x reference)
import functools

import jax
import jax.numpy as jnp
from jax import lax
from jax.experimental import pallas as pl
from jax.experimental.pallas import tpu as pltpu
from jax.experimental.pallas import tpu_sc as plsc

BATCH = 16384
FIELDS = 26
EMBED = 64

NUM_CORES = 2
NUM_SUBCORES = 16
NUM_WORKERS = NUM_CORES * NUM_SUBCORES

B_FLAT = BATCH * FIELDS
B_PER_W = B_FLAT // NUM_WORKERS
CHUNK = 1024
N_CHUNKS = B_PER_W // CHUNK


def _gather_body(idx_hbm, table_hbm, out_hbm, idx_v, rows_v, sem):
    wid = lax.axis_index("s") * NUM_CORES + lax.axis_index("c")
    base = wid * B_PER_W

    @pl.loop(0, N_CHUNKS)
    def _chunk(i):
        off = base + i * CHUNK
        pltpu.sync_copy(idx_hbm.at[pl.ds(off, CHUNK)], idx_v)
        pltpu.async_copy(table_hbm.at[idx_v], rows_v, sem).wait()
        pltpu.sync_copy(rows_v, out_hbm.at[pl.ds(off, CHUNK)])


@jax.jit
def _embed(idx_flat, weight):
    mesh = plsc.VectorSubcoreMesh(core_axis_name="c", subcore_axis_name="s")
    k = functools.partial(
        pl.kernel,
        out_type=jax.ShapeDtypeStruct((B_FLAT, EMBED), jnp.float32),
        mesh=mesh,
        scratch_types=[
            pltpu.VMEM((CHUNK,), jnp.int32),
            pltpu.VMEM((CHUNK, EMBED), jnp.float32),
            pltpu.SemaphoreType.DMA,
        ],
        compiler_params=pltpu.CompilerParams(use_tc_tiling_on_sc=False),
    )(_gather_body)
    return k(idx_flat, weight)


def kernel(input, weight):
    idx_flat = input.reshape(B_FLAT).astype(jnp.int32)
    out = _embed(idx_flat, weight)
    return out.reshape(BATCH, FIELDS, EMBED)


# speedup vs baseline: 1.0131x; 1.0131x over previous
import functools

import jax
import jax.numpy as jnp
from jax import lax
from jax.experimental import pallas as pl
from jax.experimental.pallas import tpu as pltpu
from jax.experimental.pallas import tpu_sc as plsc

BATCH = 16384
FIELDS = 26
EMBED = 64

NUM_CORES = 2
NUM_SUBCORES = 16
NUM_WORKERS = NUM_CORES * NUM_SUBCORES

B_FLAT = BATCH * FIELDS
B_PER_W = B_FLAT // NUM_WORKERS
CHUNK = 416
N_CHUNKS = B_PER_W // CHUNK
NBUF = 4


def _gather_body(idx_hbm, table_hbm, out_hbm, idx_v, rows_v, g_sems, w_sems):
    wid = lax.axis_index("s") * NUM_CORES + lax.axis_index("c")
    base = wid * B_PER_W

    pltpu.sync_copy(idx_hbm.at[pl.ds(base, B_PER_W)], idx_v)

    def idx_chunk(i):
        return idx_v.at[pl.ds(i * CHUNK, CHUNK)]

    def gather(i, b):
        pltpu.async_copy(table_hbm.at[idx_chunk(i)], rows_v.at[b], g_sems.at[b])

    def wait_gather(b):
        pltpu.make_async_copy(
            table_hbm.at[idx_chunk(0)], rows_v.at[b], g_sems.at[b]
        ).wait()

    def writeback(i, b):
        pltpu.async_copy(
            rows_v.at[b], out_hbm.at[pl.ds(base + i * CHUNK, CHUNK)], w_sems.at[b]
        )

    def wait_writeback(b):
        pltpu.make_async_copy(
            rows_v.at[b], out_hbm.at[pl.ds(base, CHUNK)], w_sems.at[b]
        ).wait()

    for b in range(NBUF):
        gather(b, b)

    @pl.loop(0, N_CHUNKS, step=NBUF)
    def _outer(i0):
        for b in range(NBUF):
            i = i0 + b
            wait_gather(b)
            writeback(i, b)

            @pl.when(i + NBUF < N_CHUNKS)
            def _refill():
                wait_writeback(b)
                gather(i + NBUF, b)

    for b in range(NBUF):
        wait_writeback(b)


@jax.jit
def _embed(idx_flat, weight):
    mesh = plsc.VectorSubcoreMesh(core_axis_name="c", subcore_axis_name="s")
    k = functools.partial(
        pl.kernel,
        out_type=jax.ShapeDtypeStruct((B_FLAT, EMBED), jnp.float32),
        mesh=mesh,
        scratch_types=[
            pltpu.VMEM((B_PER_W,), jnp.int32),
            pltpu.VMEM((NBUF, CHUNK, EMBED), jnp.float32),
            pltpu.SemaphoreType.DMA((NBUF,)),
            pltpu.SemaphoreType.DMA((NBUF,)),
        ],
        compiler_params=pltpu.CompilerParams(use_tc_tiling_on_sc=False),
    )(_gather_body)
    return k(idx_flat, weight)


def kernel(input, weight):
    idx_flat = input.reshape(B_FLAT).astype(jnp.int32)
    out = _embed(idx_flat, weight)
    return out.reshape(BATCH, FIELDS, EMBED)


# speedup vs baseline: 1.2368x; 1.2208x over previous
import functools

import jax
import jax.numpy as jnp
from jax import lax
from jax.experimental import pallas as pl
from jax.experimental.pallas import tpu as pltpu
from jax.experimental.pallas import tpu_sc as plsc

BATCH = 16384
FIELDS = 26
EMBED = 64

FIELDS_PAD = 32
EMBED_PAD = 128

NUM_CORES = 2
NUM_SUBCORES = 16
NUM_WORKERS = NUM_CORES * NUM_SUBCORES

B_PER_W = BATCH // NUM_WORKERS
NBUF = 2


def _gather_body(idx_hbm, table_hbm, out_hbm, idx_v, rows_v, g_sems, w_sems):
    wid = lax.axis_index("s") * NUM_CORES + lax.axis_index("c")
    b0 = wid * B_PER_W

    def gather(f, b):
        pltpu.sync_copy(idx_hbm.at[pl.ds(f * BATCH + b0, B_PER_W)], idx_v.at[b])
        pltpu.async_copy(table_hbm.at[idx_v.at[b]], rows_v.at[b], g_sems.at[b])

    def wait_gather(b):
        pltpu.make_async_copy(
            table_hbm.at[idx_v.at[b]], rows_v.at[b], g_sems.at[b]
        ).wait()

    def writeback(f, b):
        pltpu.async_copy(
            rows_v.at[b],
            out_hbm.at[pl.ds(b0, B_PER_W), f, pl.ds(0, EMBED)],
            w_sems.at[b],
        )

    def wait_writeback(b):
        pltpu.make_async_copy(
            rows_v.at[b],
            out_hbm.at[pl.ds(b0, B_PER_W), 0, pl.ds(0, EMBED)],
            w_sems.at[b],
        ).wait()

    for b in range(NBUF):
        gather(b, b)

    @pl.loop(0, FIELDS, step=NBUF)
    def _outer(f0):
        for b in range(NBUF):
            f = f0 + b
            wait_gather(b)
            writeback(f, b)

            @pl.when(f + NBUF < FIELDS)
            def _refill():
                wait_writeback(b)
                gather(f + NBUF, b)

    for b in range(NBUF):
        wait_writeback(b)


@jax.jit
def _embed(idx_flat, weight):
    mesh = plsc.VectorSubcoreMesh(core_axis_name="c", subcore_axis_name="s")
    k = functools.partial(
        pl.kernel,
        out_type=jax.ShapeDtypeStruct((BATCH, FIELDS_PAD, EMBED_PAD), jnp.float32),
        mesh=mesh,
        scratch_types=[
            pltpu.VMEM((NBUF, B_PER_W), jnp.int32),
            pltpu.VMEM((NBUF, B_PER_W, EMBED), jnp.float32),
            pltpu.SemaphoreType.DMA((NBUF,)),
            pltpu.SemaphoreType.DMA((NBUF,)),
        ],
        compiler_params=pltpu.CompilerParams(use_tc_tiling_on_sc=False),
    )(_gather_body)
    return k(idx_flat, weight)


def kernel(input, weight):
    idx_flat = input.astype(jnp.int32).T.reshape(BATCH * FIELDS)
    out_pad = _embed(idx_flat, weight)
    return out_pad[:, :FIELDS, :EMBED]
